# baseline (speedup 1.0000x reference)
import jax
import jax.numpy as jnp
import numpy as np
from jax import lax
from jax.experimental import pallas as pl
from jax.experimental.pallas import tpu as pltpu
from jax.experimental.pallas import tpu_sc as plsc

ZCH_N = 1000000
N_VALS = 327680
DIM = 64
NC, NS, LANES = 2, 16, 16
NW = NC * NS
PER_W = N_VALS // NW
GCH = 128
NBUF = 8
NCH2 = 2 * PER_W // GCH

BC = 32768
X1 = 21 * BC
X2 = 999424
ZPAD = 31 * BC
SCCH = 1024
NSC = (X2 - X1) // SCCH
REG1_T = X1 // NS
REG2_T = (X2 - X1) // NS
REG3_T = BC // 2 // NS
A_TAIL = X1 + (X2 - 30 * BC)


def _rowsum_body(tt_ref, out_ref):
    out_ref[...] = jnp.sum(tt_ref[...], axis=0)


def _sc_remap_body(v0_hbm, v1_hbm, tt_hbm, r0_hbm, r1_hbm, rsb_hbm,
                   vals_v, idx_v, buf_v, accb_v):
    wid = lax.axis_index("s") * NC + lax.axis_index("c")

    def feature(v_hbm, r_hbm):
        pltpu.sync_copy(v_hbm.at[pl.ds(wid * PER_W, PER_W)], vals_v)

        def mod_row(j, _):
            for k in range(GCH // 16):
                sl = pl.ds(j * GCH + k * 16, 16)
                v = vals_v[sl]
                v = v - jnp.where(v >= 2 * ZCH_N, 2 * ZCH_N, 0)
                v = v - jnp.where(v >= ZCH_N, ZCH_N, 0)
                idx_v[sl] = v
            return 0
        lax.fori_loop(0, PER_W // GCH, mod_row, 0)
        pltpu.sync_copy(idx_v, r_hbm.at[pl.ds(wid * PER_W, PER_W)])

    feature(v0_hbm, r0_hbm)
    feature(v1_hbm, r1_hbm)

    def sweep(k, _):
        m = wid + NW * k

        @pl.when(m < NSC)
        def _():
            col0 = X1 + m * SCCH
            pltpu.sync_copy(tt_hbm.at[:, pl.ds(col0, SCCH)], buf_v)

            def col_grp(l, _):
                def drow(d, a):
                    return (a + buf_v[2 * d, pl.ds(l * 16, 16)]
                            + buf_v[2 * d + 1, pl.ds(l * 16, 16)])
                acc = lax.fori_loop(0, DIM // 2, drow,
                                    jnp.zeros((16,), jnp.float32))
                accb_v[pl.ds(l * 16, 16)] = acc
                return 0
            lax.fori_loop(0, SCCH // 16, col_grp, 0)
            pltpu.sync_copy(accb_v, rsb_hbm.at[pl.ds(m * SCCH, SCCH)])
        return 0
    lax.fori_loop(0, (NSC + NW - 1) // NW, sweep, 0)


def _sc_gather_body(r0_hbm, r1_hbm, rsa_hbm, rsb_hbm, part_hbm,
                    idx_v, g_v, acc_v, rs_sh, bounce_v, *sems):
    s = lax.axis_index("s")
    wid = s * NC + lax.axis_index("c")

    for h in range(2):
        off = s * REG1_T + h * (REG1_T // 2)
        pltpu.sync_copy(rsa_hbm.at[pl.ds(off, REG1_T // 2)],
                        bounce_v.at[pl.ds(0, REG1_T // 2)])
        pltpu.sync_copy(bounce_v.at[pl.ds(0, REG1_T // 2)],
                        rs_sh.at[pl.ds(off, REG1_T // 2)])
    pltpu.sync_copy(rsb_hbm.at[pl.ds(s * REG2_T, REG2_T)],
                    bounce_v.at[pl.ds(0, REG2_T)])
    pltpu.sync_copy(bounce_v.at[pl.ds(0, REG2_T)],
                    rs_sh.at[pl.ds(X1 + s * REG2_T, REG2_T)])
    pltpu.sync_copy(rsa_hbm.at[pl.ds(A_TAIL + s * REG3_T, REG3_T)],
                    bounce_v.at[pl.ds(0, REG3_T)])
    pltpu.sync_copy(bounce_v.at[pl.ds(0, REG3_T)],
                    rs_sh.at[pl.ds(X2 + s * REG3_T, REG3_T)])

    pltpu.sync_copy(r0_hbm.at[pl.ds(wid * PER_W, PER_W)],
                    idx_v.at[pl.ds(0, PER_W)])
    pltpu.sync_copy(r1_hbm.at[pl.ds(wid * PER_W, PER_W)],
                    idx_v.at[pl.ds(PER_W, PER_W)])
    plsc.subcore_barrier()

    for b in range(NBUF):
        pltpu.async_copy(rs_sh.at[idx_v.at[pl.ds(b * GCH, GCH)]],
                         g_v.at[b], sems[b])

    def group(q, accs):
        j = q * NBUF
        a0, a1 = accs
        for b in range(NBUF):
            pltpu.make_async_copy(
                rs_sh.at[idx_v.at[pl.ds((j + b) * GCH, GCH)]],
                g_v.at[b], sems[b]).wait()
            a0 = (a0 + g_v[b, pl.ds(0, 16)] + g_v[b, pl.ds(32, 16)]
                  + g_v[b, pl.ds(64, 16)] + g_v[b, pl.ds(96, 16)])
            a1 = (a1 + g_v[b, pl.ds(16, 16)] + g_v[b, pl.ds(48, 16)]
                  + g_v[b, pl.ds(80, 16)] + g_v[b, pl.ds(112, 16)])

            @pl.when(q < NCH2 // NBUF - 1)
            def _():
                pltpu.async_copy(
                    rs_sh.at[idx_v.at[pl.ds((j + NBUF + b) * GCH, GCH)]],
                    g_v.at[b], sems[b])
        return (a0, a1)

    zero = jnp.zeros((LANES,), jnp.float32)
    a0, a1 = lax.fori_loop(0, NCH2 // NBUF, group, (zero, zero))
    acc_v[...] = a0 + a1
    pltpu.sync_copy(acc_v, part_hbm.at[wid])


@jax.jit
def kernel(values_feature_0, values_feature_0_readonly, table):
    v0 = values_feature_0
    v1 = values_feature_0_readonly

    tt = table.T
    rsa = pl.pallas_call(
        _rowsum_body,
        grid=(22,),
        in_specs=[pl.BlockSpec((DIM, BC), lambda i: (0, jnp.where(i < 21, i, 30)))],
        out_specs=pl.BlockSpec((BC,), lambda i: (i,)),
        out_shape=jax.ShapeDtypeStruct((22 * BC, ), jnp.float32),
    )(tt)

    mesh = plsc.VectorSubcoreMesh(core_axis_name="c", subcore_axis_name="s")
    r0, r1, rsb = pl.kernel(
        _sc_remap_body,
        out_type=[
            jax.ShapeDtypeStruct((N_VALS,), jnp.int32),
            jax.ShapeDtypeStruct((N_VALS,), jnp.int32),
            jax.ShapeDtypeStruct((X2 - X1,), jnp.float32),
        ],
        mesh=mesh,
        scratch_types=[
            pltpu.VMEM((PER_W,), jnp.int32),
            pltpu.VMEM((PER_W,), jnp.int32),
            pltpu.VMEM((DIM, SCCH), jnp.float32),
            pltpu.VMEM((SCCH,), jnp.float32),
        ],
    )(v0, v1, tt)

    part = pl.kernel(
        _sc_gather_body,
        out_type=jax.ShapeDtypeStruct((NW, LANES), jnp.float32),
        mesh=mesh,
        compiler_params=pltpu.CompilerParams(use_tc_tiling_on_sc=False),
        scratch_types=[
            pltpu.VMEM((2 * PER_W,), jnp.int32),
            pltpu.VMEM((NBUF, GCH), jnp.float32),
            pltpu.VMEM((LANES,), jnp.float32),
            pltpu.VMEM_SHARED((ZPAD,), jnp.float32),
            pltpu.VMEM((REG1_T // 2,), jnp.float32),
        ] + [pltpu.SemaphoreType.DMA] * NBUF,
    )(r0, r1, rsa, rsb)

    loss = part.sum() / np.float32(2 * N_VALS * DIM)
    return (loss, (r0, r1))


# speedup vs baseline: 1.4730x; 1.4730x over previous
import jax
import jax.numpy as jnp
import numpy as np
from jax import lax
from jax.experimental import pallas as pl
from jax.experimental.pallas import tpu as pltpu
from jax.experimental.pallas import tpu_sc as plsc

ZCH_N = 1000000
N_VALS = 327680
DIM = 64
NC, NS, LANES = 2, 16, 16
NW = NC * NS
PER_W = N_VALS // NW
GCH = 128
NBUF = 8
NCH2 = 2 * PER_W // GCH

BC = 32768
X1 = 21 * BC
X2 = 999424
ZPAD = 31 * BC
SCCH = 1024
NSC = (X2 - X1) // SCCH
REG1_T = X1 // NS
REG2_T = (X2 - X1) // NS
REG3_T = BC // 2 // NS
A_TAIL = X1 + (X2 - 30 * BC)


def _rowsum_body(tt_ref, out_ref):
    out_ref[...] = jnp.sum(tt_ref[...], axis=0)


def _sc_remap_body(v0_hbm, v1_hbm, tt_hbm, r0_hbm, r1_hbm, rsb_hbm,
                   vals_v, idx_v, buf_v, accb_v):
    wid = lax.axis_index("s") * NC + lax.axis_index("c")

    def feature(v_hbm, r_hbm):
        pltpu.sync_copy(v_hbm.at[pl.ds(wid * PER_W, PER_W)], vals_v)

        def mod_row(j, _):
            for k in range(GCH // 16):
                sl = pl.ds(j * GCH + k * 16, 16)
                v = vals_v[sl]
                v = v - jnp.where(v >= 2 * ZCH_N, 2 * ZCH_N, 0)
                v = v - jnp.where(v >= ZCH_N, ZCH_N, 0)
                idx_v[sl] = v
            return 0
        lax.fori_loop(0, PER_W // GCH, mod_row, 0)
        pltpu.sync_copy(idx_v, r_hbm.at[pl.ds(wid * PER_W, PER_W)])

    feature(v0_hbm, r0_hbm)
    feature(v1_hbm, r1_hbm)

    def sweep(k, _):
        m = wid + NW * k

        @pl.when(m < NSC)
        def _():
            col0 = X1 + m * SCCH
            pltpu.sync_copy(tt_hbm.at[:, pl.ds(col0, SCCH)], buf_v)

            def col_grp(l, _):
                sl = pl.ds(l * 16, 16)
                e0 = buf_v[0, sl]
                e1 = buf_v[1, sl]
                for d in range(2, DIM, 2):
                    e0 = e0 + buf_v[d, sl]
                    e1 = e1 + buf_v[d + 1, sl]
                accb_v[sl] = e0 + e1
                return 0
            lax.fori_loop(0, SCCH // 16, col_grp, 0)
            pltpu.sync_copy(accb_v, rsb_hbm.at[pl.ds(m * SCCH, SCCH)])
        return 0
    lax.fori_loop(0, (NSC + NW - 1) // NW, sweep, 0)


def _sc_gather_body(r0_hbm, r1_hbm, rsa_hbm, rsb_hbm, part_hbm,
                    idx_v, g_v, acc_v, rs_sh, bounce_v, *sems):
    s = lax.axis_index("s")
    wid = s * NC + lax.axis_index("c")

    for h in range(2):
        off = s * REG1_T + h * (REG1_T // 2)
        pltpu.sync_copy(rsa_hbm.at[pl.ds(off, REG1_T // 2)],
                        bounce_v.at[pl.ds(0, REG1_T // 2)])
        pltpu.sync_copy(bounce_v.at[pl.ds(0, REG1_T // 2)],
                        rs_sh.at[pl.ds(off, REG1_T // 2)])
    pltpu.sync_copy(rsb_hbm.at[pl.ds(s * REG2_T, REG2_T)],
                    bounce_v.at[pl.ds(0, REG2_T)])
    pltpu.sync_copy(bounce_v.at[pl.ds(0, REG2_T)],
                    rs_sh.at[pl.ds(X1 + s * REG2_T, REG2_T)])
    pltpu.sync_copy(rsa_hbm.at[pl.ds(A_TAIL + s * REG3_T, REG3_T)],
                    bounce_v.at[pl.ds(0, REG3_T)])
    pltpu.sync_copy(bounce_v.at[pl.ds(0, REG3_T)],
                    rs_sh.at[pl.ds(X2 + s * REG3_T, REG3_T)])

    pltpu.sync_copy(r0_hbm.at[pl.ds(wid * PER_W, PER_W)],
                    idx_v.at[pl.ds(0, PER_W)])
    pltpu.sync_copy(r1_hbm.at[pl.ds(wid * PER_W, PER_W)],
                    idx_v.at[pl.ds(PER_W, PER_W)])
    plsc.subcore_barrier()

    for b in range(NBUF):
        pltpu.async_copy(rs_sh.at[idx_v.at[pl.ds(b * GCH, GCH)]],
                         g_v.at[b], sems[b])

    def group(q, accs):
        j = q * NBUF
        a0, a1 = accs
        for b in range(NBUF):
            pltpu.make_async_copy(
                rs_sh.at[idx_v.at[pl.ds((j + b) * GCH, GCH)]],
                g_v.at[b], sems[b]).wait()
            a0 = (a0 + g_v[b, pl.ds(0, 16)] + g_v[b, pl.ds(32, 16)]
                  + g_v[b, pl.ds(64, 16)] + g_v[b, pl.ds(96, 16)])
            a1 = (a1 + g_v[b, pl.ds(16, 16)] + g_v[b, pl.ds(48, 16)]
                  + g_v[b, pl.ds(80, 16)] + g_v[b, pl.ds(112, 16)])

            @pl.when(q < NCH2 // NBUF - 1)
            def _():
                pltpu.async_copy(
                    rs_sh.at[idx_v.at[pl.ds((j + NBUF + b) * GCH, GCH)]],
                    g_v.at[b], sems[b])
        return (a0, a1)

    zero = jnp.zeros((LANES,), jnp.float32)
    a0, a1 = lax.fori_loop(0, NCH2 // NBUF, group, (zero, zero))
    acc_v[...] = a0 + a1
    pltpu.sync_copy(acc_v, part_hbm.at[wid])


@jax.jit
def kernel(values_feature_0, values_feature_0_readonly, table):
    v0 = values_feature_0
    v1 = values_feature_0_readonly

    tt = table.T
    rsa = pl.pallas_call(
        _rowsum_body,
        grid=(22,),
        in_specs=[pl.BlockSpec((DIM, BC), lambda i: (0, jnp.where(i < 21, i, 30)))],
        out_specs=pl.BlockSpec((BC,), lambda i: (i,)),
        out_shape=jax.ShapeDtypeStruct((22 * BC, ), jnp.float32),
    )(tt)

    mesh = plsc.VectorSubcoreMesh(core_axis_name="c", subcore_axis_name="s")
    r0, r1, rsb = pl.kernel(
        _sc_remap_body,
        out_type=[
            jax.ShapeDtypeStruct((N_VALS,), jnp.int32),
            jax.ShapeDtypeStruct((N_VALS,), jnp.int32),
            jax.ShapeDtypeStruct((X2 - X1,), jnp.float32),
        ],
        mesh=mesh,
        scratch_types=[
            pltpu.VMEM((PER_W,), jnp.int32),
            pltpu.VMEM((PER_W,), jnp.int32),
            pltpu.VMEM((DIM, SCCH), jnp.float32),
            pltpu.VMEM((SCCH,), jnp.float32),
        ],
    )(v0, v1, tt)

    part = pl.kernel(
        _sc_gather_body,
        out_type=jax.ShapeDtypeStruct((NW, LANES), jnp.float32),
        mesh=mesh,
        compiler_params=pltpu.CompilerParams(use_tc_tiling_on_sc=False),
        scratch_types=[
            pltpu.VMEM((2 * PER_W,), jnp.int32),
            pltpu.VMEM((NBUF, GCH), jnp.float32),
            pltpu.VMEM((LANES,), jnp.float32),
            pltpu.VMEM_SHARED((ZPAD,), jnp.float32),
            pltpu.VMEM((REG1_T // 2,), jnp.float32),
        ] + [pltpu.SemaphoreType.DMA] * NBUF,
    )(r0, r1, rsa, rsb)

    loss = part.sum() / np.float32(2 * N_VALS * DIM)
    return (loss, (r0, r1))


# speedup vs baseline: 1.6608x; 1.1275x over previous
import jax
import jax.numpy as jnp
import numpy as np
from jax import lax
from jax.experimental import pallas as pl
from jax.experimental.pallas import tpu as pltpu
from jax.experimental.pallas import tpu_sc as plsc

ZCH_N = 1000000
N_VALS = 327680
DIM = 64
NC, NS, LANES = 2, 16, 16
NW = NC * NS
PER_W = N_VALS // NW
GCH = 128
NG = PER_W // GCH

BC = 32768
NBLK = (ZCH_N + BC - 1) // BC
ZPAD = NBLK * BC


def _rowsum_body(tt_ref, out_ref):
    out_ref[...] = jnp.sum(tt_ref[...], axis=0)


NBUF = 8
NG2 = 2 * NG
RCH = ZPAD // NS


def _sc_remap_body(v0_hbm, v1_hbm, r0_hbm, r1_hbm, vals_v, idx_v):
    wid = lax.axis_index("s") * NC + lax.axis_index("c")

    def feature(v_hbm, r_hbm):
        pltpu.sync_copy(v_hbm.at[pl.ds(wid * PER_W, PER_W)], vals_v)

        def mod_row(j, _):
            for k in range(GCH // 16):
                sl = pl.ds(k * 16, 16)
                v = vals_v[pl.ds(j * GCH + k * 16, 16)]
                v = v - jnp.where(v >= 2 * ZCH_N, 2 * ZCH_N, 0)
                v = v - jnp.where(v >= ZCH_N, ZCH_N, 0)
                idx_v[j, sl] = v
            return 0
        lax.fori_loop(0, NG, mod_row, 0)
        pltpu.sync_copy(idx_v, r_hbm.at[wid])

    feature(v0_hbm, r0_hbm)
    feature(v1_hbm, r1_hbm)


def _sc_gather_body(r0_hbm, r1_hbm, rs_hbm, part_hbm, idx_v, g_v, acc_v, rs_sh,
                    rs_bounce_v, *sems):
    s = lax.axis_index("s")
    wid = s * NC + lax.axis_index("c")
    for h in range(2):
        off = s * RCH + h * (RCH // 2)
        pltpu.sync_copy(rs_hbm.at[pl.ds(off, RCH // 2)], rs_bounce_v)
        pltpu.sync_copy(rs_bounce_v, rs_sh.at[pl.ds(off, RCH // 2)])
    pltpu.sync_copy(r0_hbm.at[wid], idx_v.at[pl.ds(0, NG)])
    pltpu.sync_copy(r1_hbm.at[wid], idx_v.at[pl.ds(NG, NG)])
    plsc.subcore_barrier()

    for b in range(NBUF):
        pltpu.async_copy(rs_sh.at[idx_v.at[b]], g_v.at[b], sems[b])

    def group(q, accs):
        j = q * NBUF
        a0, a1 = accs
        for b in range(NBUF):
            pltpu.make_async_copy(rs_sh.at[idx_v.at[j + b]], g_v.at[b],
                                  sems[b]).wait()
            a0 = (a0 + g_v[b, pl.ds(0, 16)] + g_v[b, pl.ds(32, 16)]
                  + g_v[b, pl.ds(64, 16)] + g_v[b, pl.ds(96, 16)])
            a1 = (a1 + g_v[b, pl.ds(16, 16)] + g_v[b, pl.ds(48, 16)]
                  + g_v[b, pl.ds(80, 16)] + g_v[b, pl.ds(112, 16)])

            @pl.when(q < NG2 // NBUF - 1)
            def _():
                pltpu.async_copy(rs_sh.at[idx_v.at[j + NBUF + b]],
                                 g_v.at[b], sems[b])
        return (a0, a1)

    zero = jnp.zeros((LANES,), jnp.float32)
    a0, a1 = lax.fori_loop(0, NG2 // NBUF, group, (zero, zero))
    acc_v[...] = a0 + a1
    pltpu.sync_copy(acc_v, part_hbm.at[wid])


@jax.jit
def kernel(values_feature_0, values_feature_0_readonly, table):
    v0 = values_feature_0
    v1 = values_feature_0_readonly

    tt = table.T
    rowsum = pl.pallas_call(
        _rowsum_body,
        grid=(NBLK,),
        in_specs=[pl.BlockSpec((DIM, BC), lambda i: (0, i))],
        out_specs=pl.BlockSpec((BC,), lambda i: (i,)),
        out_shape=jax.ShapeDtypeStruct((ZPAD,), jnp.float32),
    )(tt)

    mesh = plsc.VectorSubcoreMesh(core_axis_name="c", subcore_axis_name="s")
    r0_3d, r1_3d = pl.kernel(
        _sc_remap_body,
        out_type=[
            jax.ShapeDtypeStruct((NW, NG, GCH), jnp.int32),
            jax.ShapeDtypeStruct((NW, NG, GCH), jnp.int32),
        ],
        mesh=mesh,
        compiler_params=pltpu.CompilerParams(use_tc_tiling_on_sc=False),
        scratch_types=[
            pltpu.VMEM((PER_W,), jnp.int32),
            pltpu.VMEM((NG, GCH), jnp.int32),
        ],
    )(v0, v1)

    part = pl.kernel(
        _sc_gather_body,
        out_type=jax.ShapeDtypeStruct((NW, LANES), jnp.float32),
        mesh=mesh,
        compiler_params=pltpu.CompilerParams(use_tc_tiling_on_sc=False),
        scratch_types=[
            pltpu.VMEM((NG2, GCH), jnp.int32),
            pltpu.VMEM((NBUF, GCH), jnp.float32),
            pltpu.VMEM((LANES,), jnp.float32),
            pltpu.VMEM_SHARED((ZPAD,), jnp.float32),
            pltpu.VMEM((RCH // 2,), jnp.float32),
        ] + [pltpu.SemaphoreType.DMA] * NBUF,
    )(r0_3d, r1_3d, rowsum)

    loss = part.sum() / np.float32(2 * N_VALS * DIM)
    return (loss, (r0_3d.reshape(-1), r1_3d.reshape(-1)))
